# Initial kernel scaffold; baseline (speedup 1.0000x reference)
import jax
import jax.numpy as jnp
from jax.experimental import pallas as pl


def kernel(users, pos_items, neg_items, lap_rows, lap_cols, lap_vals, user_emb, item_emb, W1, b1, W2, b2):
    raise NotImplementedError("write your pallas kernel here")


# speedup vs baseline: 3.5897x; 3.5897x over previous
import functools

import jax
import jax.numpy as jnp
from jax import lax
from jax.experimental import pallas as pl
from jax.experimental.pallas import tpu as pltpu
from jax.experimental.pallas import tpu_sc as plsc

D = 64
DH = 32
CHUNK = 25000
CP = 25088
TS = CP // 16
ZB = 224
G = 512
NCHUNK = 4


def _prep_edges(a, half, ep):
    a2 = a.reshape(2, half)
    a2 = jnp.pad(a2, ((0, 0), (0, ep - half)))
    return a2.reshape(2, ep // 128, 128)


def _spmm(rows_h, cols_h, vals_h, ego_lo, ego_hi, jblk):
    mesh = plsc.VectorSubcoreMesh(core_axis_name="c", subcore_axis_name="s")

    @functools.partial(
        pl.kernel,
        out_type=jax.ShapeDtypeStruct((NCHUNK, 2, CP, DH), jnp.float32),
        mesh=mesh,
        scratch_types=[
            pltpu.VMEM((4, 128), jnp.int32),
            pltpu.VMEM((4, 128), jnp.int32),
            pltpu.VMEM((4, 128), jnp.float32),
            pltpu.VMEM((4, 128), jnp.int32),
            pltpu.VMEM((G, DH), jnp.float32),
            pltpu.VMEM((ZB, DH), jnp.float32),
            pltpu.VMEM_SHARED((CP, DH), jnp.float32),
            pltpu.SemaphoreType.DMA,
        ],
        compiler_params=pltpu.CompilerParams(use_tc_tiling_on_sc=False),
    )
    def k(rows_hbm, cols_hbm, vals_hbm, elo_hbm, ehi_hbm, out_hbm,
          rbuf, cbuf, vbuf, ibuf, gbuf, zbuf, acc, sem):
        c = lax.axis_index("c")
        s = lax.axis_index("s")

        def zz(i, carry):
            for jj in range(DH // 16):
                zbuf[i, pl.ds(jj * 16, 16)] = jnp.zeros((16,), jnp.float32)
            return carry
        lax.fori_loop(0, ZB, zz, 0)

        for p in range(2):
            for f in range(2):
                ch = 2 * p + c
                lo = ch * CHUNK
                ego_hbm = elo_hbm if f == 0 else ehi_hbm
                for kq in range(TS // ZB):
                    pltpu.sync_copy(zbuf, acc.at[pl.ds(s * TS + kq * ZB, ZB)])
                plsc.subcore_barrier()

                def blk(j, carry):
                    g0 = j * 64 + s * 4
                    pltpu.sync_copy(rows_hbm.at[p, pl.ds(g0, 4)], rbuf)
                    pltpu.sync_copy(cols_hbm.at[p, pl.ds(g0, 4)], cbuf)
                    pltpu.sync_copy(vals_hbm.at[p, pl.ds(g0, 4)], vbuf)
                    cps = [
                        pltpu.async_copy(ego_hbm.at[cbuf.at[q]],
                                         gbuf.at[pl.ds(q * 128, 128)], sem)
                        for q in range(4)
                    ]
                    for cp in cps:
                        cp.wait()
                    for q in range(4):
                        def msk(i, carry2):
                            sl = pl.ds(i * 16, 16)
                            rv = rbuf[q, sl]
                            vv = vbuf[q, sl]
                            m = (rv >= lo) & (rv < lo + CHUNK)
                            vbuf[q, sl] = jnp.where(m, vv, 0.0)
                            ibuf[q, sl] = jnp.where(m, rv - lo, 0)
                            return carry2
                        lax.fori_loop(0, 8, msk, 0)
                    for q in range(4):
                        def erow16(i, carry2):
                            sv = vbuf[q, pl.ds(i * 16, 16)]
                            for k16 in range(16):
                                sc = sv[k16]
                                r = q * 128 + i * 16 + k16
                                for jj in range(DH // 16):
                                    sl = pl.ds(jj * 16, 16)
                                    gbuf[r, sl] = gbuf[r, sl] * sc
                            return carry2
                        lax.fori_loop(0, 8, erow16, 0)
                    for q in range(4):
                        pltpu.sync_copy(gbuf.at[pl.ds(q * 128, 128)],
                                        acc.at[ibuf.at[q]], add=True)
                    return carry
                lax.fori_loop(0, jblk, blk, 0)

                plsc.subcore_barrier()
                pltpu.sync_copy(acc.at[pl.ds(s * TS, TS)],
                                out_hbm.at[ch, f, pl.ds(s * TS, TS)])

    return k(rows_h, cols_h, vals_h, ego_lo, ego_hi)


def _dense(msg_p, ego_s, w1, b1, w2, b2, n_nodes):
    br = 1000
    jgrid = CHUNK // br

    def body(mlo_ref, mhi_ref, elo_ref, ehi_ref,
             w1_ref, b1_ref, w2_ref, b2_ref, eo_ref, no_ref):
        egos = jnp.concatenate([elo_ref[0], ehi_ref[0]], axis=1)
        msg = jnp.concatenate([mlo_ref[0, 0], mhi_ref[0, 0]], axis=1)
        side = msg + egos
        dn = (((1,), (0,)), ((), ()))
        a = lax.dot_general(side, w1_ref[...], dn,
                            preferred_element_type=jnp.float32) + b1_ref[...]
        se = jnp.maximum(a, 0.2 * a)
        b = lax.dot_general(egos * side, w2_ref[...], dn,
                            preferred_element_type=jnp.float32) + b2_ref[...]
        be = jnp.maximum(b, 0.2 * b)
        e = se + be
        eo_ref[0] = e[:, :DH]
        eo_ref[1] = e[:, DH:]
        nr = jnp.sqrt(jnp.sum(e * e, axis=1, keepdims=True)) + 1e-12
        no_ref[...] = e / nr

    return pl.pallas_call(
        body,
        grid=(NCHUNK, jgrid),
        in_specs=[
            pl.BlockSpec((1, 1, br, DH), lambda p, j: (p, 0, j, 0)),
            pl.BlockSpec((1, 1, br, DH), lambda p, j: (p, 1, j, 0)),
            pl.BlockSpec((1, br, DH), lambda p, j: (0, p * jgrid + j, 0)),
            pl.BlockSpec((1, br, DH), lambda p, j: (1, p * jgrid + j, 0)),
            pl.BlockSpec((D, D), lambda p, j: (0, 0)),
            pl.BlockSpec((1, D), lambda p, j: (0, 0)),
            pl.BlockSpec((D, D), lambda p, j: (0, 0)),
            pl.BlockSpec((1, D), lambda p, j: (0, 0)),
        ],
        out_specs=[
            pl.BlockSpec((2, br, DH), lambda p, j: (0, p * jgrid + j, 0)),
            pl.BlockSpec((br, D), lambda p, j: (p * jgrid + j, 0)),
        ],
        out_shape=(jax.ShapeDtypeStruct((2, n_nodes, DH), jnp.float32),
                   jax.ShapeDtypeStruct((n_nodes, D), jnp.float32)),
    )(msg_p, msg_p, ego_s, ego_s, w1, b1.reshape(1, D), w2, b2.reshape(1, D))


def _final_gather(users, pos_items, neg_items, tables, n_user, batch):
    mesh = plsc.VectorSubcoreMesh(core_axis_name="c", subcore_axis_name="s")
    per_w = batch // 32
    width = D * len(tables)

    @functools.partial(
        pl.kernel,
        out_type=tuple(jax.ShapeDtypeStruct((batch, width), jnp.float32)
                       for _ in range(3)),
        mesh=mesh,
        scratch_types=[
            pltpu.VMEM((per_w,), jnp.int32),
            pltpu.VMEM((per_w, D), jnp.float32),
            pltpu.SemaphoreType.DMA,
        ],
        compiler_params=pltpu.CompilerParams(use_tc_tiling_on_sc=False),
    )
    def k(users_h, pos_h, neg_h, t0, t1, t2, t3, o0, o1, o2,
          ibuf, gbuf, sem):
        c = lax.axis_index("c")
        s = lax.axis_index("s")
        wid = s * 2 + c
        base = wid * per_w
        for src, off, out in ((users_h, 0, o0), (pos_h, n_user, o1),
                              (neg_h, n_user, o2)):
            pltpu.sync_copy(src.at[pl.ds(base, per_w)], ibuf)
            if off:
                def addoff(i, carry):
                    sl = pl.ds(i * 16, 16)
                    ibuf[sl] = ibuf[sl] + off
                    return carry
                lax.fori_loop(0, per_w // 16, addoff, 0)
            for t, tab in enumerate((t0, t1, t2, t3)):
                pltpu.async_copy(tab.at[ibuf], gbuf, sem).wait()
                pltpu.sync_copy(gbuf, out.at[pl.ds(base, per_w),
                                             pl.ds(t * D, D)])

    return k(users, pos_items, neg_items, *tables)


def kernel(users, pos_items, neg_items, lap_rows, lap_cols, lap_vals,
           user_emb, item_emb, W1, b1, W2, b2):
    n_user = user_emb.shape[0]
    n_nodes = n_user + item_emb.shape[0]
    nnz = lap_rows.shape[0]
    half = nnz // 2
    batch = users.shape[0]
    jblk = -(-half // (16 * 4 * 128))
    ep = jblk * 16 * 4 * 128

    rows_h = _prep_edges(lap_rows, half, ep)
    cols_h = _prep_edges(lap_cols, half, ep)
    vals_h = _prep_edges(lap_vals, half, ep)

    ego0 = jnp.concatenate([user_emb, item_emb], axis=0)
    ego_s = jnp.stack([ego0[:, :DH], ego0[:, DH:]], axis=0)
    tables = [ego0]
    for l in range(len(W1)):
        msg_p = _spmm(rows_h, cols_h, vals_h, ego_s[0], ego_s[1], jblk)
        ego_s, nrm = _dense(msg_p, ego_s, W1[l], b1[l], W2[l], b2[l], n_nodes)
        tables.append(nrm)
    return _final_gather(users, pos_items, neg_items, tables, n_user, batch)
